# baseline (speedup 1.0000x reference)
import functools

import jax
import jax.numpy as jnp
from jax import lax
from jax.experimental import pallas as pl
from jax.experimental.pallas import tpu as pltpu
from jax.experimental.pallas import tpu_sc as plsc

D = 256
KMAX = 64
L = 512
SH = 1016
NE = 2 * KMAX + 1

_mesh = plsc.VectorSubcoreMesh(core_axis_name="c", subcore_axis_name="s")


@functools.partial(
    pl.kernel,
    mesh=_mesh,
    out_type=jax.ShapeDtypeStruct((L, L, D), jnp.float32),
    scratch_types=[
        pltpu.VMEM((NE, D), jnp.float32),
        pltpu.VMEM((64, D), jnp.float32),
        pltpu.VMEM_SHARED((4 * SH, D), jnp.float32),
        pltpu.SemaphoreType.DMA,
    ],
)
def _rel_pos_sc(emb, out, emb_vm, stage_vm, tabs, wsem):
    s = lax.axis_index("s")
    c = lax.axis_index("c")

    chunk = jnp.minimum(s * 64, SH - 64)

    pltpu.sync_copy(emb, emb_vm)
    for p in range(4):
        shift = (c * 4 + p) - (L - 1 - KMAX)

        def copy_row(t, _):
            row = jnp.clip(chunk + t + shift, 0, 2 * KMAX)
            for l in range(D // 16):
                stage_vm[t, pl.ds(l * 16, 16)] = emb_vm[row, pl.ds(l * 16, 16)]
            return _

        lax.fori_loop(0, 64, copy_row, None, unroll=False)
        pltpu.sync_copy(stage_vm, tabs.at[pl.ds(p * SH + chunk, 64)])
    plsc.subcore_barrier()

    copies = []
    for p in range(4):
        r = c * 4 + p
        for q in range(2):
            m = s * 2 + q
            i = (L - 1) - r - 8 * m
            copies.append(pltpu.async_copy(
                tabs.at[pl.ds(p * SH + 8 * m, L)], out.at[i], wsem))
    for cp in copies:
        cp.wait()


_RPB = 8


def _rel_pos_tc_body(emb_ref, out_ref, b_ref):
    g = pl.program_id(0)

    @pl.when(g == 0)
    def _():
        for r in range(8):
            w0 = L - 1 - KMAX - r
            b_ref[r, 0:w0] = jnp.broadcast_to(emb_ref[0:1], (w0, D))
            b_ref[r, w0:w0 + NE] = emb_ref[...]
            b_ref[r, w0 + NE:] = jnp.broadcast_to(
                emb_ref[NE - 1:NE], (SH - (w0 + NE), D))

    for rr in range(_RPB):
        r = (7 - rr) % 8
        start = pl.multiple_of((L - 1) - rr - r - g * _RPB, 8)
        out_ref[rr] = b_ref[r, pl.ds(start, L)]


_rel_pos_tc = pl.pallas_call(
    _rel_pos_tc_body,
    grid=(L // _RPB,),
    in_specs=[pl.BlockSpec((NE, D), lambda g: (0, 0))],
    out_specs=pl.BlockSpec((_RPB, L, D), lambda g: (g, 0, 0)),
    out_shape=jax.ShapeDtypeStruct((L, L, D), jnp.float32),
    scratch_shapes=[pltpu.VMEM((8, SH, D), jnp.float32)],
)


def _rel_pos_tc_tail_body(emb_ref, vin_ref, out_ref, b_ref):
    del vin_ref
    _rel_pos_tc_body(emb_ref, out_ref, b_ref)


_rel_pos_tc_tail = pl.pallas_call(
    _rel_pos_tc_tail_body,
    grid=(256 // _RPB,),
    in_specs=[
        pl.BlockSpec((NE, D), lambda g: (0, 0)),
        pl.BlockSpec(memory_space=pl.ANY),
    ],
    out_specs=pl.BlockSpec((_RPB, L, D), lambda g: (g, 0, 0)),
    out_shape=jax.ShapeDtypeStruct((L, L, D), jnp.float32),
    scratch_shapes=[pltpu.VMEM((8, SH, D), jnp.float32)],
    input_output_aliases={1: 0},
)


def kernel(emb_k, emb_v, length_q):
    del length_q
    v_part = _rel_pos_sc(emb_v)
    out_v = _rel_pos_tc_tail(emb_v, v_part)
    return out_v, out_v


# speedup vs baseline: 1.7201x; 1.7201x over previous
import functools

import jax
import jax.numpy as jnp
from jax import lax
from jax.experimental import pallas as pl
from jax.experimental.pallas import tpu as pltpu
from jax.experimental.pallas import tpu_sc as plsc

D = 256
KMAX = 64
L = 512
SH = 1016
NE = 2 * KMAX + 1

_mesh = plsc.VectorSubcoreMesh(core_axis_name="c", subcore_axis_name="s")


@functools.partial(
    pl.kernel,
    mesh=_mesh,
    out_type=jax.ShapeDtypeStruct((L, L, D), jnp.float32),
    scratch_types=[
        pltpu.VMEM((NE, D), jnp.float32),
        pltpu.VMEM((64, D), jnp.float32),
        pltpu.VMEM_SHARED((4 * SH, D), jnp.float32),
        pltpu.SemaphoreType.DMA,
    ],
)
def _rel_pos_sc(emb, out, emb_vm, stage_vm, tabs, wsem):
    s = lax.axis_index("s")
    c = lax.axis_index("c")

    chunk = jnp.minimum(s * 64, SH - 64)

    pltpu.sync_copy(emb, emb_vm)
    for p in range(4):
        shift = (c * 4 + p) - (L - 1 - KMAX)

        def copy_row(t, _):
            row = jnp.clip(chunk + t + shift, 0, 2 * KMAX)
            for l in range(D // 16):
                stage_vm[t, pl.ds(l * 16, 16)] = emb_vm[row, pl.ds(l * 16, 16)]
            return _

        lax.fori_loop(0, 64, copy_row, None, unroll=False)
        pltpu.sync_copy(stage_vm, tabs.at[pl.ds(p * SH + chunk, 64)])
    plsc.subcore_barrier()

    copies = []
    for p in range(4):
        r = c * 4 + p
        m = s
        i = (L - 1) - r - 8 * m
        copies.append(pltpu.async_copy(
            tabs.at[pl.ds(p * SH + 8 * m, L)], out.at[i], wsem))
    for cp in copies:
        cp.wait()


_RPB = 8


def _rel_pos_tc_body(emb_ref, out_ref, b_ref):
    g = pl.program_id(0)

    @pl.when(g == 0)
    def _():
        for r in range(8):
            w0 = L - 1 - KMAX - r
            b_ref[r, 0:w0] = jnp.broadcast_to(emb_ref[0:1], (w0, D))
            b_ref[r, w0:w0 + NE] = emb_ref[...]
            b_ref[r, w0 + NE:] = jnp.broadcast_to(
                emb_ref[NE - 1:NE], (SH - (w0 + NE), D))

    for rr in range(_RPB):
        r = (7 - rr) % 8
        start = pl.multiple_of((L - 1) - rr - r - g * _RPB, 8)
        out_ref[rr] = b_ref[r, pl.ds(start, L)]


_rel_pos_tc = pl.pallas_call(
    _rel_pos_tc_body,
    grid=(L // _RPB,),
    in_specs=[pl.BlockSpec((NE, D), lambda g: (0, 0))],
    out_specs=pl.BlockSpec((_RPB, L, D), lambda g: (g, 0, 0)),
    out_shape=jax.ShapeDtypeStruct((L, L, D), jnp.float32),
    scratch_shapes=[pltpu.VMEM((8, SH, D), jnp.float32)],
)


def _rel_pos_tc_tail_body(emb_ref, vin_ref, out_ref, b_ref):
    del vin_ref
    _rel_pos_tc_body(emb_ref, out_ref, b_ref)


_rel_pos_tc_tail = pl.pallas_call(
    _rel_pos_tc_tail_body,
    grid=(384 // _RPB,),
    in_specs=[
        pl.BlockSpec((NE, D), lambda g: (0, 0)),
        pl.BlockSpec(memory_space=pl.ANY),
    ],
    out_specs=pl.BlockSpec((_RPB, L, D), lambda g: (g, 0, 0)),
    out_shape=jax.ShapeDtypeStruct((L, L, D), jnp.float32),
    scratch_shapes=[pltpu.VMEM((8, SH, D), jnp.float32)],
    input_output_aliases={1: 0},
)


def kernel(emb_k, emb_v, length_q):
    del length_q
    v_part = _rel_pos_sc(emb_v)
    out_k = _rel_pos_tc(emb_k)
    out_v = _rel_pos_tc_tail(emb_v, v_part)
    return out_k, out_v
